# baseline (speedup 1.0000x reference)
import functools

import jax
import jax.numpy as jnp
from jax import lax
from jax.experimental import pallas as pl
from jax.experimental.pallas import tpu as pltpu
from jax.experimental.pallas import tpu_sc as plsc

EMB = 64
HID = 128
OUT = 128
B = 16384
VOCAB = 1000000

_TBLK = 65536
_TSHIFT = _TBLK.bit_length() - 1
_Q = _TBLK // 4

_NGRID = (VOCAB + _TBLK - 1) // _TBLK
_NQUAD_PAD = _NGRID * _Q

_TCH = 1024


def _pack_pair(hi_f32, lo_f32):
    hi = lax.bitcast_convert_type(hi_f32.astype(jnp.bfloat16), jnp.uint16)
    lo = lax.bitcast_convert_type(lo_f32.astype(jnp.bfloat16), jnp.uint16)
    w = jnp.left_shift(hi.astype(jnp.uint32), 16) | lo.astype(jnp.uint32)
    return lax.bitcast_convert_type(w, jnp.float32)


def _transpose_body(xt_ref, o_ref):
    for j in range(_Q // _TCH):
        lo = j * _TCH
        ab = _pack_pair(
            xt_ref[:, lo : lo + _TCH],
            xt_ref[:, _Q + lo : _Q + lo + _TCH],
        )
        cd = _pack_pair(
            xt_ref[:, 2 * _Q + lo : 2 * _Q + lo + _TCH],
            xt_ref[:, 3 * _Q + lo : 3 * _Q + lo + _TCH],
        )
        o_ref[lo : lo + _TCH, 0:EMB] = ab.T
        o_ref[lo : lo + _TCH, EMB : 2 * EMB] = cd.T


def _tc_quadize(table_t):
    return pl.pallas_call(
        _transpose_body,
        grid=(_NGRID,),
        in_specs=[pl.BlockSpec((EMB, _TBLK), lambda i: (0, i))],
        out_specs=pl.BlockSpec((_Q, 2 * EMB), lambda i: (i, 0)),
        out_shape=jax.ShapeDtypeStruct((_NQUAD_PAD, 2 * EMB), jnp.float32),
    )(table_t)


def _make_sc_gather(D2, batch):
    try:
        info = plsc.get_sparse_core_info()
        NC, NS = info.num_cores, info.num_subcores
    except Exception:
        NC, NS = 2, 16
    NW = NC * NS
    assert batch % (8 * NW) == 0
    b_per_w = batch // NW
    mesh = plsc.VectorSubcoreMesh(
        core_axis_name="c", subcore_axis_name="s", num_cores=NC
    )

    @functools.partial(
        pl.kernel,
        mesh=mesh,
        out_type=jax.ShapeDtypeStruct((batch, D2), jnp.float32),
        scratch_types=[
            pltpu.VMEM((b_per_w,), jnp.int32),
            pltpu.VMEM((b_per_w, D2), jnp.float32),
            pltpu.SemaphoreType.DMA,
        ],
    )
    def gather_k(table_hbm, idx_hbm, out_hbm, idx_v, rows_v, sem):
        wid = lax.axis_index("s") * NC + lax.axis_index("c")
        base = wid * b_per_w
        pltpu.sync_copy(idx_hbm.at[pl.ds(base, b_per_w)], idx_v)
        pltpu.async_copy(table_hbm.at[idx_v], rows_v, sem).wait()
        pltpu.sync_copy(rows_v, out_hbm.at[pl.ds(base, b_per_w)])

    return gather_k


def _mlp_body(x_ref, qlo_ref, qhi_ref, w1t2_ref, b1_ref, w2t_ref, b2_ref, o_ref):
    u = lax.bitcast_convert_type(x_ref[...], jnp.int32)
    qlo = qlo_ref[...]
    sel_u = jnp.where(qlo == 0, u & jnp.int32(-65536), jnp.left_shift(u, 16))
    xsel = lax.bitcast_convert_type(sel_u, jnp.float32)
    qhi = qhi_ref[...]
    col = lax.broadcasted_iota(jnp.int32, (1, 2 * EMB), 1)
    keep = (col < EMB) == (qhi == 0.0)
    x2 = jnp.where(keep, xsel, 0.0)
    h = jnp.dot(x2, w1t2_ref[...], preferred_element_type=jnp.float32)
    h = h + b1_ref[...]
    o_ref[...] = (
        jnp.dot(h, w2t_ref[...], preferred_element_type=jnp.float32) + b2_ref[...]
    )


def _tc_mlp(x, qlo, qhi, w1t2, b1, w2t, b2):
    blk = 4096
    grid = (B // blk,)
    return pl.pallas_call(
        _mlp_body,
        grid=grid,
        in_specs=[
            pl.BlockSpec((blk, 2 * EMB), lambda i: (i, 0)),
            pl.BlockSpec((blk, 1), lambda i: (i, 0)),
            pl.BlockSpec((blk, 1), lambda i: (i, 0)),
            pl.BlockSpec((2 * EMB, HID), lambda i: (0, 0)),
            pl.BlockSpec((1, HID), lambda i: (0, 0)),
            pl.BlockSpec((HID, OUT), lambda i: (0, 0)),
            pl.BlockSpec((1, OUT), lambda i: (0, 0)),
        ],
        out_specs=pl.BlockSpec((blk, OUT), lambda i: (i, 0)),
        out_shape=jax.ShapeDtypeStruct((B, OUT), jnp.float32),
    )(x, qlo, qhi, w1t2, b1, w2t, b2)


_sc_gather_cache = {}


def kernel(input_emotion, table, W1, b1, W2, b2):
    if "g" not in _sc_gather_cache:
        _sc_gather_cache["g"] = _make_sc_gather(2 * EMB, B)
    table_quads = _tc_quadize(table.T)
    r = input_emotion
    quad_idx = jnp.bitwise_or(
        jnp.left_shift(lax.shift_right_logical(r, _TSHIFT), _TSHIFT - 2),
        jnp.bitwise_and(r, _Q - 1),
    )
    sub = jnp.bitwise_and(lax.shift_right_logical(r, _TSHIFT - 2), 3)
    qhi = lax.shift_right_logical(sub, 1).astype(jnp.float32).reshape(B, 1)
    qlo = jnp.bitwise_and(sub, 1).reshape(B, 1)
    gathered = _sc_gather_cache["g"](table_quads, quad_idx)
    w1t2 = jnp.concatenate([W1.T, W1.T], axis=0)
    out = _tc_mlp(
        gathered,
        qlo,
        qhi,
        w1t2,
        b1.reshape(1, HID),
        W2.T,
        b2.reshape(1, OUT),
    )
    return out.reshape(1, B, OUT)


# speedup vs baseline: 1.0208x; 1.0208x over previous
import functools

import jax
import jax.numpy as jnp
from jax import lax
from jax.experimental import pallas as pl
from jax.experimental.pallas import tpu as pltpu
from jax.experimental.pallas import tpu_sc as plsc

EMB = 64
HID = 128
OUT = 128
B = 16384
VOCAB = 1000000

_TBLK = 32768
_TSHIFT = _TBLK.bit_length() - 1
_Q = _TBLK // 4

_NGRID = (VOCAB + _TBLK - 1) // _TBLK
_NQUAD_PAD = _NGRID * _Q

_TCH = 1024


def _pack_pair(hi_f32, lo_f32):
    hi = lax.bitcast_convert_type(hi_f32.astype(jnp.bfloat16), jnp.uint16)
    lo = lax.bitcast_convert_type(lo_f32.astype(jnp.bfloat16), jnp.uint16)
    w = jnp.left_shift(hi.astype(jnp.uint32), 16) | lo.astype(jnp.uint32)
    return lax.bitcast_convert_type(w, jnp.float32)


def _transpose_body(xt_ref, o_ref):
    for j in range(_Q // _TCH):
        lo = j * _TCH
        ab = _pack_pair(
            xt_ref[:, lo : lo + _TCH],
            xt_ref[:, _Q + lo : _Q + lo + _TCH],
        )
        cd = _pack_pair(
            xt_ref[:, 2 * _Q + lo : 2 * _Q + lo + _TCH],
            xt_ref[:, 3 * _Q + lo : 3 * _Q + lo + _TCH],
        )
        o_ref[lo : lo + _TCH, 0:EMB] = ab.T
        o_ref[lo : lo + _TCH, EMB : 2 * EMB] = cd.T


def _tc_quadize(table_t):
    return pl.pallas_call(
        _transpose_body,
        grid=(_NGRID,),
        in_specs=[pl.BlockSpec((EMB, _TBLK), lambda i: (0, i))],
        out_specs=pl.BlockSpec((_Q, 2 * EMB), lambda i: (i, 0)),
        out_shape=jax.ShapeDtypeStruct((_NQUAD_PAD, 2 * EMB), jnp.float32),
    )(table_t)


def _make_sc_gather(D2, batch):
    try:
        info = plsc.get_sparse_core_info()
        NC, NS = info.num_cores, info.num_subcores
    except Exception:
        NC, NS = 2, 16
    NW = NC * NS
    assert batch % (8 * NW) == 0
    b_per_w = batch // NW
    mesh = plsc.VectorSubcoreMesh(
        core_axis_name="c", subcore_axis_name="s", num_cores=NC
    )

    @functools.partial(
        pl.kernel,
        mesh=mesh,
        out_type=jax.ShapeDtypeStruct((batch, D2), jnp.float32),
        scratch_types=[
            pltpu.VMEM((b_per_w,), jnp.int32),
            pltpu.VMEM((b_per_w, D2), jnp.float32),
            pltpu.SemaphoreType.DMA,
        ],
    )
    def gather_k(table_hbm, idx_hbm, out_hbm, idx_v, rows_v, sem):
        wid = lax.axis_index("s") * NC + lax.axis_index("c")
        base = wid * b_per_w
        pltpu.sync_copy(idx_hbm.at[pl.ds(base, b_per_w)], idx_v)
        pltpu.async_copy(table_hbm.at[idx_v], rows_v, sem).wait()
        pltpu.sync_copy(rows_v, out_hbm.at[pl.ds(base, b_per_w)])

    return gather_k


def _mlp_body(x_ref, qlo_ref, qhi_ref, w1t2_ref, b1_ref, w2t_ref, b2_ref, o_ref):
    u = lax.bitcast_convert_type(x_ref[...], jnp.int32)
    qlo = qlo_ref[...]
    sel_u = jnp.where(qlo == 0, u & jnp.int32(-65536), jnp.left_shift(u, 16))
    xsel = lax.bitcast_convert_type(sel_u, jnp.float32)
    qhi = qhi_ref[...]
    col = lax.broadcasted_iota(jnp.int32, (1, 2 * EMB), 1)
    keep = (col < EMB) == (qhi == 0.0)
    x2 = jnp.where(keep, xsel, 0.0)
    h = jnp.dot(x2, w1t2_ref[...], preferred_element_type=jnp.float32)
    h = h + b1_ref[...]
    o_ref[...] = (
        jnp.dot(h, w2t_ref[...], preferred_element_type=jnp.float32) + b2_ref[...]
    )


def _tc_mlp(x, qlo, qhi, w1t2, b1, w2t, b2):
    blk = 4096
    grid = (B // blk,)
    return pl.pallas_call(
        _mlp_body,
        grid=grid,
        in_specs=[
            pl.BlockSpec((blk, 2 * EMB), lambda i: (i, 0)),
            pl.BlockSpec((blk, 1), lambda i: (i, 0)),
            pl.BlockSpec((blk, 1), lambda i: (i, 0)),
            pl.BlockSpec((2 * EMB, HID), lambda i: (0, 0)),
            pl.BlockSpec((1, HID), lambda i: (0, 0)),
            pl.BlockSpec((HID, OUT), lambda i: (0, 0)),
            pl.BlockSpec((1, OUT), lambda i: (0, 0)),
        ],
        out_specs=pl.BlockSpec((blk, OUT), lambda i: (i, 0)),
        out_shape=jax.ShapeDtypeStruct((B, OUT), jnp.float32),
    )(x, qlo, qhi, w1t2, b1, w2t, b2)


_sc_gather_cache = {}


def kernel(input_emotion, table, W1, b1, W2, b2):
    if "g" not in _sc_gather_cache:
        _sc_gather_cache["g"] = _make_sc_gather(2 * EMB, B)
    table_quads = _tc_quadize(table.T)
    r = input_emotion
    quad_idx = jnp.bitwise_or(
        jnp.left_shift(lax.shift_right_logical(r, _TSHIFT), _TSHIFT - 2),
        jnp.bitwise_and(r, _Q - 1),
    )
    sub = jnp.bitwise_and(lax.shift_right_logical(r, _TSHIFT - 2), 3)
    qhi = lax.shift_right_logical(sub, 1).astype(jnp.float32).reshape(B, 1)
    qlo = jnp.bitwise_and(sub, 1).reshape(B, 1)
    gathered = _sc_gather_cache["g"](table_quads, quad_idx)
    w1t2 = jnp.concatenate([W1.T, W1.T], axis=0)
    out = _tc_mlp(
        gathered,
        qlo,
        qhi,
        w1t2,
        b1.reshape(1, HID),
        W2.T,
        b2.reshape(1, OUT),
    )
    return out.reshape(1, B, OUT)


# speedup vs baseline: 1.0624x; 1.0407x over previous
import functools

import jax
import jax.numpy as jnp
from jax import lax
from jax.experimental import pallas as pl
from jax.experimental.pallas import tpu as pltpu
from jax.experimental.pallas import tpu_sc as plsc

EMB = 64
HID = 128
OUT = 128
B = 16384
VOCAB = 1000000

_TBLK = 32768
_TSHIFT = _TBLK.bit_length() - 1
_Q = _TBLK // 4

_NGRID = (VOCAB + _TBLK - 1) // _TBLK
_NQUAD_PAD = _NGRID * _Q

_TCH = 1024


def _pack_pair(hi_f32, lo_f32):
    hi = lax.bitcast_convert_type(hi_f32.astype(jnp.bfloat16), jnp.uint16)
    lo = lax.bitcast_convert_type(lo_f32.astype(jnp.bfloat16), jnp.uint16)
    w = jnp.left_shift(hi.astype(jnp.uint32), 16) | lo.astype(jnp.uint32)
    return lax.bitcast_convert_type(w, jnp.float32)


def _transpose_body(xt_ref, o_ref):
    for j in range(_Q // _TCH):
        lo = j * _TCH
        ab = _pack_pair(
            xt_ref[:, lo : lo + _TCH],
            xt_ref[:, _Q + lo : _Q + lo + _TCH],
        )
        cd = _pack_pair(
            xt_ref[:, 2 * _Q + lo : 2 * _Q + lo + _TCH],
            xt_ref[:, 3 * _Q + lo : 3 * _Q + lo + _TCH],
        )
        o_ref[lo : lo + _TCH, 0:EMB] = ab.T
        o_ref[lo : lo + _TCH, EMB : 2 * EMB] = cd.T


def _tc_quadize(table_t):
    return pl.pallas_call(
        _transpose_body,
        grid=(_NGRID,),
        in_specs=[pl.BlockSpec((EMB, _TBLK), lambda i: (0, i))],
        out_specs=pl.BlockSpec((_Q, 2 * EMB), lambda i: (i, 0)),
        out_shape=jax.ShapeDtypeStruct((_NQUAD_PAD, 2 * EMB), jnp.float32),
    )(table_t)


def _make_sc_gather(D2, batch):
    try:
        info = plsc.get_sparse_core_info()
        NC, NS = info.num_cores, info.num_subcores
    except Exception:
        NC, NS = 2, 16
    NW = NC * NS
    assert batch % (8 * NW) == 0
    b_per_w = batch // NW
    mesh = plsc.VectorSubcoreMesh(
        core_axis_name="c", subcore_axis_name="s", num_cores=NC
    )

    @functools.partial(
        pl.kernel,
        mesh=mesh,
        out_type=jax.ShapeDtypeStruct((batch, D2), jnp.float32),
        scratch_types=[
            pltpu.VMEM((b_per_w,), jnp.int32),
            pltpu.VMEM((b_per_w, D2), jnp.float32),
            pltpu.SemaphoreType.DMA,
        ],
    )
    def gather_k(table_hbm, idx_hbm, out_hbm, idx_v, rows_v, sem):
        wid = lax.axis_index("s") * NC + lax.axis_index("c")
        base = wid * b_per_w
        pltpu.sync_copy(idx_hbm.at[pl.ds(base, b_per_w)], idx_v)
        pltpu.async_copy(table_hbm.at[idx_v], rows_v, sem).wait()
        pltpu.sync_copy(rows_v, out_hbm.at[pl.ds(base, b_per_w)])

    return gather_k


def _mlp_body(x_ref, sub_ref, w1t2_ref, b1_ref, w2t_ref, b2_ref, o_ref):
    u = lax.bitcast_convert_type(x_ref[...], jnp.int32)
    sub = sub_ref[...]
    qlo = jnp.bitwise_and(sub, 1)
    sel_u = jnp.where(qlo == 0, u & jnp.int32(-65536), jnp.left_shift(u, 16))
    xsel = lax.bitcast_convert_type(sel_u, jnp.float32)
    qhi = lax.shift_right_logical(sub, 1)
    col = lax.broadcasted_iota(jnp.int32, (1, 2 * EMB), 1)
    keep = (col < EMB) == (qhi == 0)
    x2 = jnp.where(keep, xsel, 0.0)
    h = jnp.dot(x2, w1t2_ref[...], preferred_element_type=jnp.float32)
    h = h + b1_ref[...]
    o_ref[...] = (
        jnp.dot(h, w2t_ref[...], preferred_element_type=jnp.float32) + b2_ref[...]
    )


def _tc_mlp(x, sub, w1t2, b1, w2t, b2):
    blk = 4096
    grid = (B // blk,)
    return pl.pallas_call(
        _mlp_body,
        grid=grid,
        in_specs=[
            pl.BlockSpec((blk, 2 * EMB), lambda i: (i, 0)),
            pl.BlockSpec((blk, 1), lambda i: (i, 0)),
            pl.BlockSpec((2 * EMB, HID), lambda i: (0, 0)),
            pl.BlockSpec((1, HID), lambda i: (0, 0)),
            pl.BlockSpec((HID, OUT), lambda i: (0, 0)),
            pl.BlockSpec((1, OUT), lambda i: (0, 0)),
        ],
        out_specs=pl.BlockSpec((blk, OUT), lambda i: (i, 0)),
        out_shape=jax.ShapeDtypeStruct((B, OUT), jnp.float32),
    )(x, sub, w1t2, b1, w2t, b2)


_sc_gather_cache = {}


def kernel(input_emotion, table, W1, b1, W2, b2):
    if "g" not in _sc_gather_cache:
        _sc_gather_cache["g"] = _make_sc_gather(2 * EMB, B)
    table_quads = _tc_quadize(table.T)
    r = input_emotion
    quad_idx = jnp.bitwise_or(
        jnp.left_shift(lax.shift_right_logical(r, _TSHIFT), _TSHIFT - 2),
        jnp.bitwise_and(r, _Q - 1),
    )
    sub = jnp.bitwise_and(lax.shift_right_logical(r, _TSHIFT - 2), 3).reshape(B, 1)
    gathered = _sc_gather_cache["g"](table_quads, quad_idx)
    w1t2 = jnp.concatenate([W1.T, W1.T], axis=0)
    out = _tc_mlp(
        gathered,
        sub,
        w1t2,
        b1.reshape(1, HID),
        W2.T,
        b2.reshape(1, OUT),
    )
    return out.reshape(1, B, OUT)
